# baseline (speedup 1.0000x reference)
import jax
import jax.numpy as jnp
from jax.experimental import pallas as pl


def _gcn_block(adj_a_ref, adj_b_ref, x_ref, w_ref, b_ref, out_ref):
    bm = adj_a_ref.shape[0]
    acc_a = jnp.dot(adj_a_ref[...], x_ref[...],
                    preferred_element_type=jnp.float32)
    out_ref[:bm, :] = jnp.dot(acc_a, w_ref[...],
                              preferred_element_type=jnp.float32) + b_ref[...]
    acc_b = jnp.dot(adj_b_ref[...], x_ref[...],
                    preferred_element_type=jnp.float32)
    out_ref[bm:, :] = jnp.dot(acc_b, w_ref[...],
                              preferred_element_type=jnp.float32) + b_ref[...]


def kernel(adj, x, W, b):
    n, k = adj.shape
    d_in = x.shape[1]
    d_out = W.shape[1]
    bm = 200
    grid = (n // (2 * bm),)
    out = pl.pallas_call(
        _gcn_block,
        grid=grid,
        in_specs=[
            pl.BlockSpec((bm, k), lambda i: (2 * i, 0)),
            pl.BlockSpec((bm, k), lambda i: (2 * i + 1, 0)),
            pl.BlockSpec((k, d_in), lambda i: (0, 0)),
            pl.BlockSpec((d_in, d_out), lambda i: (0, 0)),
            pl.BlockSpec((1, d_out), lambda i: (0, 0)),
        ],
        out_specs=pl.BlockSpec((2 * bm, d_out), lambda i: (i, 0)),
        out_shape=jax.ShapeDtypeStruct((n, d_out), jnp.float32),
    )(adj, adj, x, W, b.reshape(1, d_out))
    return out


# speedup vs baseline: 1.0872x; 1.0872x over previous
import jax
import jax.numpy as jnp
from jax.experimental import pallas as pl
from jax.experimental.pallas import tpu as pltpu


def _gcn_block(adj_a_ref, adj_b_ref, x_ref, w_ref, b_ref, out_ref, h_ref):
    @pl.when(pl.program_id(0) == 0)
    def _():
        h_ref[...] = jnp.dot(x_ref[...], w_ref[...],
                             preferred_element_type=jnp.float32)
    bm = adj_a_ref.shape[0]
    out_ref[:bm, :] = jnp.dot(adj_a_ref[...], h_ref[...],
                              preferred_element_type=jnp.float32) + b_ref[...]
    out_ref[bm:, :] = jnp.dot(adj_b_ref[...], h_ref[...],
                              preferred_element_type=jnp.float32) + b_ref[...]


def kernel(adj, x, W, b):
    n, k = adj.shape
    d_in = x.shape[1]
    d_out = W.shape[1]
    bm = 200
    if n % (2 * bm):
        bm = n

    if bm == n:
        def _single(adj_ref, x_ref, w_ref, b_ref, out_ref):
            h = jnp.dot(x_ref[...], w_ref[...],
                        preferred_element_type=jnp.float32)
            out_ref[...] = jnp.dot(adj_ref[...], h,
                                   preferred_element_type=jnp.float32) + b_ref[...]
        return pl.pallas_call(
            _single,
            out_shape=jax.ShapeDtypeStruct((n, d_out), jnp.float32),
        )(adj, x, W, b.reshape(1, d_out))

    grid = (n // (2 * bm),)
    out = pl.pallas_call(
        _gcn_block,
        grid=grid,
        in_specs=[
            pl.BlockSpec((bm, k), lambda i: (2 * i, 0)),
            pl.BlockSpec((bm, k), lambda i: (2 * i + 1, 0)),
            pl.BlockSpec((k, d_in), lambda i: (0, 0)),
            pl.BlockSpec((d_in, d_out), lambda i: (0, 0)),
            pl.BlockSpec((1, d_out), lambda i: (0, 0)),
        ],
        out_specs=pl.BlockSpec((2 * bm, d_out), lambda i: (i, 0)),
        out_shape=jax.ShapeDtypeStruct((n, d_out), jnp.float32),
        scratch_shapes=[pltpu.VMEM((k, d_out), jnp.float32)],
    )(adj, adj, x, W, b.reshape(1, d_out))
    return out
